# baseline (speedup 1.0000x reference)
import functools

import jax
import jax.numpy as jnp
from jax import lax
from jax.experimental import pallas as pl
from jax.experimental.pallas import tpu as pltpu
from jax.experimental.pallas import tpu_sc as plsc

NC = 2
NS = 16
NW = NC * NS

CW = 128
DEPTH = 4


def _make_gather(V, D, NJ, NI):
    nq = NJ * (NI // CW)
    q_per_w = nq // NW
    ic_per_j = NI // CW
    mesh = plsc.VectorSubcoreMesh(core_axis_name="c", subcore_axis_name="s")

    @functools.partial(
        pl.kernel,
        out_type=jax.ShapeDtypeStruct((NJ, D, NI), jnp.float32),
        mesh=mesh,
        scratch_types=(
            [pltpu.VMEM((CW,), jnp.int32) for _ in range(DEPTH)]
            + [pltpu.VMEM((CW, D), jnp.float32) for _ in range(DEPTH)]
            + [pltpu.VMEM((D, CW), jnp.float32) for _ in range(DEPTH)]
            + [pltpu.SemaphoreType.DMA for _ in range(3 * DEPTH)]
        ),
        compiler_params=pltpu.CompilerParams(
            use_tc_tiling_on_sc=False, needs_layout_passes=False),
    )
    def k(table_hbm, idx_hbm, out_hbm, *bufs):
        idxb = bufs[:DEPTH]
        rows = bufs[DEPTH:2 * DEPTH]
        tbuf = bufs[2 * DEPTH:3 * DEPTH]
        isem = bufs[3 * DEPTH:4 * DEPTH]
        gsem = bufs[4 * DEPTH:5 * DEPTH]
        wsem = bufs[5 * DEPTH:]
        wid = lax.axis_index("s") * NC + lax.axis_index("c")
        q0 = wid * q_per_w
        iota16 = lax.iota(jnp.int32, 16)

        def idx_copy(q, s):
            j = q // ic_per_j
            i0 = (q % ic_per_j) * CW
            return pltpu.make_async_copy(
                idx_hbm.at[j, pl.ds(i0, CW)], idxb[s], isem[s])

        def write_copy(q, s):
            j = q // ic_per_j
            i0 = (q % ic_per_j) * CW
            return pltpu.make_async_copy(
                tbuf[s], out_hbm.at[j, :, pl.ds(i0, CW)], wsem[s])

        def gather_copy(s):
            return pltpu.make_async_copy(table_hbm.at[idxb[s]], rows[s], gsem[s])

        def transpose(s):
            @pl.loop(0, D)
            def _d(d):
                dvec = jnp.full((16,), 0, jnp.int32) + d
                for g in range(CW // 16):
                    vals = plsc.load_gather(rows[s], [g * 16 + iota16, dvec])
                    tbuf[s][d, pl.ds(g * 16, 16)] = vals

        for b in range(DEPTH):
            idx_copy(q0 + b, b).start()
        for b in range(2):
            idx_copy(q0 + b, b).wait()
            gather_copy(b).start()

        @pl.loop(0, q_per_w, step=DEPTH)
        def _group(t):
            for b in range(DEPTH):
                tq = t + b
                q = q0 + tq
                s = b
                gather_copy(s).wait()
                @pl.when(tq + DEPTH < q_per_w)
                def _nexti():
                    idx_copy(q + DEPTH, s).start()
                @pl.when(tq >= DEPTH)
                def _drainw():
                    write_copy(q - DEPTH, s).wait()
                transpose(s)
                write_copy(q, s).start()
                s2 = (b + 2) % DEPTH
                @pl.when(tq + 2 < q_per_w)
                def _nextg():
                    idx_copy(q + 2, s2).wait()
                    gather_copy(s2).start()

        for b in range(DEPTH):
            tq = q_per_w - DEPTH + b
            write_copy(q0 + tq, tq % DEPTH).wait()

    return k


@jax.jit
def kernel(sparse_table, indices):
    n0, n1 = indices.shape
    V, D = sparse_table.shape
    idx_t = indices.T.astype(jnp.int32)
    out_t = _make_gather(V, D, n1, n0)(sparse_table, idx_t)
    return jnp.transpose(out_t, (2, 0, 1))
